# baseline (speedup 1.0000x reference)
import functools

import jax
import jax.numpy as jnp
from jax import lax
from jax.experimental import pallas as pl
from jax.experimental.pallas import tpu as pltpu
from jax.experimental.pallas import tpu_sc as plsc

V = 1000
V3 = 3 * V
VP = 3008
D = 128
B = 1024
P = 64
C = 3
L = 16

NC = 2
NS = 16
NW = NC * NS

BC = B // NW
IG = 8
NIG = BC // IG

_mesh = plsc.VectorSubcoreMesh(core_axis_name="c", subcore_axis_name="s")


@functools.partial(
    pl.kernel,
    out_type=jax.ShapeDtypeStruct((B, P, D), jnp.float32),
    mesh=_mesh,
    compiler_params=pltpu.CompilerParams(
        use_tc_tiling_on_sc=False, needs_layout_passes=False),
    scratch_types=[
        pltpu.VMEM((2, IG, P * C), jnp.int32),
        pltpu.VMEM((3, C, P), jnp.int32),
        pltpu.VMEM((3, P, D), jnp.bfloat16),
        pltpu.VMEM((3, P, D), jnp.float32),
        pltpu.SemaphoreType.DMA((2,)),
        pltpu.SemaphoreType.DMA((3,)),
        pltpu.SemaphoreType.DMA((3,)),
    ],
)
def _bow_kernel(tab_hbm, idx_hbm, out_hbm, idx_v, cidx_v, rows_v,
                out_v, idx_sem, rows_sem, out_sem):
    sid = lax.axis_index("s")
    wid = sid * NC + lax.axis_index("c")
    b0 = wid * BC

    def idx_copy(blk, k):
        return pltpu.make_async_copy(
            idx_hbm.at[pl.ds(b0 + blk * IG, IG)], idx_v.at[k],
            idx_sem.at[k])

    def out_copy(i, ob):
        return pltpu.make_async_copy(
            out_v.at[ob], out_hbm.at[b0 + i], out_sem.at[ob])

    def rows_copy(c, buf):
        return pltpu.make_async_copy(
            tab_hbm.at[cidx_v.at[buf, c]], rows_v.at[buf],
            rows_sem.at[buf])

    idx_copy(0, 0).start()
    idx_copy(1, 1).start()

    lane3 = lax.iota(jnp.int32, 16) * C

    def step(i, carry):
        @pl.when(i < BC)
        def _():
            blk = i // IG
            @pl.when(lax.rem(i, IG) == 0)
            def _():
                idx_copy(blk, lax.rem(blk, 2)).wait()
            buf = lax.rem(i, 3)
            iref = idx_v.at[lax.rem(blk, 2), lax.rem(i, IG)]
            for c in range(C):
                for q in range(P // L):
                    iv = plsc.load_gather(iref, [lane3 + (q * L * C + c)])
                    if c:
                        iv = iv + c * V
                    cidx_v[buf, c, pl.ds(q * L, L)] = iv
            rows_copy(0, buf).start()
            @pl.when((lax.rem(i, IG) == IG - 1) & (blk + 2 < NIG))
            def _():
                idx_copy(blk + 2, lax.rem(blk, 2)).start()
        @pl.when((i >= 1) & (i <= BC))
        def _():
            buf = lax.rem(i - 1, 3)
            rows_copy(0, buf).wait()
            rows_copy(1, buf).start(add=True)
            rows_copy(2, buf).start(add=True)
        @pl.when(i >= 2)
        def _():
            j = i - 2
            buf = lax.rem(j, 3)
            @pl.when(j >= 3)
            def _():
                out_copy(j - 3, buf).wait()
            rows_copy(1, buf).wait()
            rows_copy(2, buf).wait()
            for p in range(P):
                for q in range(D // (2 * L)):
                    r = rows_v[buf, p, pl.ds(q * 2 * L, 2 * L)]
                    lo, hi = plsc.unpack(
                        r, format=plsc.PackFormat.INTERLEAVED)
                    out_v[buf, p, pl.ds(q * L, L)] = lo
                    out_v[buf, p, pl.ds(D // 2 + q * L, L)] = hi
            out_copy(j, buf).start()
        return carry

    lax.fori_loop(0, BC + 2, step, None)

    out_copy(BC - 3, lax.rem(BC - 3, 3)).wait()
    out_copy(BC - 2, lax.rem(BC - 2, 3)).wait()
    out_copy(BC - 1, lax.rem(BC - 1, 3)).wait()


def kernel(inputs, embedding):
    b, h, w, c = inputs.shape
    u = lax.bitcast_convert_type(embedding, jnp.uint32)
    rne = (u + 0x7FFF + ((u >> 16) & 1)) >> 16
    word = (rne[:, D // 2:] << 16) | rne[:, :D // 2]
    inter = lax.bitcast_convert_type(
        word.astype(jnp.uint32), jnp.bfloat16).reshape(V3, D)
    idx = inputs.reshape(b, h * w * c).astype(jnp.int32)
    out = _bow_kernel(inter, idx)
    return jnp.transpose(out.reshape(b, h, w, D), (0, 3, 1, 2))


# speedup vs baseline: 1.0831x; 1.0831x over previous
import functools

import jax
import jax.numpy as jnp
from jax import lax
from jax.experimental import pallas as pl
from jax.experimental.pallas import tpu as pltpu
from jax.experimental.pallas import tpu_sc as plsc

V = 1000
V3 = 3 * V
VP = 3008
D = 128
B = 1024
P = 64
C = 3
L = 16

NC = 2
NS = 16
NW = NC * NS

BC = B // NW
IG = 8
NIG = BC // IG

_mesh = plsc.VectorSubcoreMesh(core_axis_name="c", subcore_axis_name="s")


@functools.partial(
    pl.kernel,
    out_type=jax.ShapeDtypeStruct((B, P, D), jnp.float32),
    mesh=_mesh,
    compiler_params=pltpu.CompilerParams(
        use_tc_tiling_on_sc=False, needs_layout_passes=False),
    scratch_types=[
        pltpu.VMEM_SHARED((VP, D), jnp.bfloat16),
        pltpu.VMEM((2, IG, P * C), jnp.int32),
        pltpu.VMEM((3, C, P), jnp.int32),
        pltpu.VMEM((3, P, D), jnp.bfloat16),
        pltpu.VMEM((3, P, D), jnp.float32),
        pltpu.SemaphoreType.DMA((2,)),
        pltpu.SemaphoreType.DMA((3,)),
        pltpu.SemaphoreType.DMA((3,)),
    ],
)
def _bow_kernel(tab_hbm, idx_hbm, out_hbm, tab_sh, idx_v, cidx_v, rows_v,
                out_v, idx_sem, rows_sem, out_sem):
    sid = lax.axis_index("s")
    wid = sid * NC + lax.axis_index("c")
    b0 = wid * BC

    def idx_copy(blk, k):
        return pltpu.make_async_copy(
            idx_hbm.at[pl.ds(b0 + blk * IG, IG)], idx_v.at[k],
            idx_sem.at[k])

    def out_copy(i, ob):
        return pltpu.make_async_copy(
            out_v.at[ob], out_hbm.at[b0 + i], out_sem.at[ob])

    def rows_copy(c, buf):
        return pltpu.make_async_copy(
            tab_sh.at[cidx_v.at[buf, c]], rows_v.at[buf],
            rows_sem.at[buf])

    idx_copy(0, 0).start()
    idx_copy(1, 1).start()
    rpt = VP // NS
    pltpu.sync_copy(tab_hbm.at[pl.ds(sid * rpt, rpt)],
                    tab_sh.at[pl.ds(sid * rpt, rpt)])
    plsc.subcore_barrier()

    lane3 = lax.iota(jnp.int32, 16) * C

    def step(i, carry):
        @pl.when(i < BC)
        def _():
            blk = i // IG
            @pl.when(lax.rem(i, IG) == 0)
            def _():
                idx_copy(blk, lax.rem(blk, 2)).wait()
            buf = lax.rem(i, 3)
            iref = idx_v.at[lax.rem(blk, 2), lax.rem(i, IG)]
            for c in range(C):
                for q in range(P // L):
                    iv = plsc.load_gather(iref, [lane3 + (q * L * C + c)])
                    if c:
                        iv = iv + c * V
                    cidx_v[buf, c, pl.ds(q * L, L)] = iv
            rows_copy(0, buf).start()
            @pl.when((lax.rem(i, IG) == IG - 1) & (blk + 2 < NIG))
            def _():
                idx_copy(blk + 2, lax.rem(blk, 2)).start()
        @pl.when((i >= 1) & (i <= BC))
        def _():
            buf = lax.rem(i - 1, 3)
            rows_copy(0, buf).wait()
            rows_copy(1, buf).start(add=True)
            rows_copy(2, buf).start(add=True)
        @pl.when(i >= 2)
        def _():
            j = i - 2
            buf = lax.rem(j, 3)
            @pl.when(j >= 3)
            def _():
                out_copy(j - 3, buf).wait()
            rows_copy(1, buf).wait()
            rows_copy(2, buf).wait()
            for p in range(P):
                for q in range(D // (2 * L)):
                    r = rows_v[buf, p, pl.ds(q * 2 * L, 2 * L)]
                    lo, hi = plsc.unpack(
                        r, format=plsc.PackFormat.INTERLEAVED)
                    out_v[buf, p, pl.ds(q * L, L)] = lo
                    out_v[buf, p, pl.ds(D // 2 + q * L, L)] = hi
            out_copy(j, buf).start()
        return carry

    lax.fori_loop(0, BC + 2, step, None)

    out_copy(BC - 3, lax.rem(BC - 3, 3)).wait()
    out_copy(BC - 2, lax.rem(BC - 2, 3)).wait()
    out_copy(BC - 1, lax.rem(BC - 1, 3)).wait()


def kernel(inputs, embedding):
    b, h, w, c = inputs.shape
    u = lax.bitcast_convert_type(embedding, jnp.uint32)
    rne = (u + 0x7FFF + ((u >> 16) & 1)) >> 16
    word = (rne[:, D // 2:] << 16) | rne[:, :D // 2]
    inter = lax.bitcast_convert_type(
        word.astype(jnp.uint32), jnp.bfloat16).reshape(V3, D)
    inter = jnp.pad(inter, ((0, VP - V3), (0, 0)))
    idx = inputs.reshape(b, h * w * c).astype(jnp.int32)
    out = _bow_kernel(inter, idx)
    return jnp.transpose(out.reshape(b, h, w, D), (0, 3, 1, 2))
